# baseline (speedup 1.0000x reference)
import functools

import jax
import jax.numpy as jnp
from jax import lax
from jax.experimental import pallas as pl
from jax.experimental.pallas import tpu as pltpu
from jax.experimental.pallas import tpu_sc as plsc

_B = 16384
_F = 26
_V = 100000
_D = 16
_C = 13
_OUT = _C + _F * _D
_N = 128

_VT = 1536
_NFULL = _V // _VT
_VLAST = 99968 - _VT
_VTAIL = 99968
_NCHUNK = _NFULL + 1


def _make_repack_kernel():
    info = plsc.get_sparse_core_info()
    nw = info.num_cores * info.num_subcores
    n_units = _F * _NCHUNK

    mesh = plsc.VectorSubcoreMesh(core_axis_name="c", subcore_axis_name="s")

    @functools.partial(
        pl.kernel,
        mesh=mesh,
        out_type=jax.ShapeDtypeStruct((_F * _V * _D,), jnp.float32),
        scratch_types=[
            pltpu.VMEM((_D, _VT), jnp.float32),
            pltpu.VMEM((_D, _VT), jnp.float32),
            pltpu.VMEM((_VT * _D,), jnp.float32),
            pltpu.VMEM((_VT * _D,), jnp.float32),
            pltpu.VMEM((32, _D), jnp.float32),
            pltpu.SemaphoreType.DMA,
            pltpu.SemaphoreType.DMA,
            pltpu.SemaphoreType.DMA,
            pltpu.SemaphoreType.DMA,
        ],
        compiler_params=pltpu.CompilerParams(
            use_tc_tiling_on_sc=True, needs_layout_passes=False
        ),
    )
    def k(wt_hbm, tail_hbm, out_hbm, slab0, slab1, rows0, rows1, tail_v,
          isem0, isem1, osem0, osem1):
        wid = lax.axis_index("s") * info.num_cores + lax.axis_index("c")
        lane = lax.iota(jnp.int32, 16)
        lane16 = lane * _D
        n_mine = (n_units - 1 - wid) // nw + 1

        def unit_coords(t):
            u = wid + t * nw
            f = u // _NCHUNK
            ci = u % _NCHUNK
            vc = jnp.where(ci == _NFULL, _VLAST, ci * _VT)
            return f, vc

        def fire_in(t, slab, isem):
            f, vc = unit_coords(t)
            pltpu.async_copy(wt_hbm.at[f, :, pl.ds(vc, _VT)], slab, isem)

        fire_in(0, slab0, isem0)

        def unit(t, slab, rows, other_slab, isem, other_isem, osem):
            @pl.when(t < n_mine)
            def _():
                f, vc = unit_coords(t)
                pltpu.make_async_copy(
                    wt_hbm.at[0, :, pl.ds(0, _VT)], slab, isem
                ).wait()

                @pl.when(t + 1 < n_mine)
                def _():
                    fire_in(t + 1, other_slab, other_isem)

                @pl.when(t >= 2)
                def _():
                    pltpu.make_async_copy(
                        rows, out_hbm.at[pl.ds(0, _VT * _D)], osem
                    ).wait()

                def seg(s, _):
                    for half in range(2):
                        s2 = s * 2 + half
                        base = lane16 + s2 * (16 * _D)
                        for d in range(_D):
                            v16 = slab[d, pl.ds(s2 * 16, 16)]
                            plsc.store_scatter(rows, [base + d], v16)
                    return ()

                lax.fori_loop(0, _VT // 32, seg, ())
                pltpu.async_copy(
                    rows,
                    out_hbm.at[pl.ds((f * _V + vc) * _D, _VT * _D)],
                    osem,
                )

        def pair(p, _):
            unit(2 * p, slab0, rows0, slab1, isem0, isem1, osem0)
            unit(2 * p + 1, slab1, rows1, slab0, isem1, isem0, osem1)
            return ()

        lax.fori_loop(0, (n_units // nw + 2) // 2, pair, ())
        for rows, osem in ((rows0, osem0), (rows1, osem1)):
            pltpu.make_async_copy(
                rows, out_hbm.at[pl.ds(0, _VT * _D)], osem
            ).wait()

        @pl.when(wid < _F)
        def _():
            f = wid
            pltpu.sync_copy(tail_hbm.at[f], tail_v)

            def trow(i, _):
                for j in range(4):
                    r = i * 4 + j
                    rows0[pl.ds(r * _D, _D)] = tail_v[r, pl.ds(0, _D)]
                return ()

            lax.fori_loop(0, 8, trow, ())
            pltpu.sync_copy(
                rows0.at[pl.ds(0, 32 * _D)],
                out_hbm.at[pl.ds((f * _V + _VTAIL) * _D, 32 * _D)],
            )

    return k


def _make_lookup_kernel():
    info = plsc.get_sparse_core_info()
    nw = info.num_cores * info.num_subcores
    nb = _B // nw
    n_sub = nb // _N

    mesh = plsc.VectorSubcoreMesh(core_axis_name="c", subcore_axis_name="s")

    @functools.partial(
        pl.kernel,
        mesh=mesh,
        out_type=jax.ShapeDtypeStruct((_B, _OUT), jnp.float32),
        scratch_types=[
            pltpu.VMEM((_N * _F,), jnp.int32),
            pltpu.VMEM((_F, _N), jnp.int32),
            pltpu.VMEM((_F * _N, _D), jnp.float32),
            pltpu.VMEM((_N, 16), jnp.float32),
            pltpu.VMEM((_N, _OUT), jnp.float32),
            pltpu.SemaphoreType.DMA,
            pltpu.SemaphoreType.DMA,
        ],
        compiler_params=pltpu.CompilerParams(
            use_tc_tiling_on_sc=False, needs_layout_passes=False
        ),
    )
    def k(cont_hbm, cat_hbm, w_hbm, out_hbm, cat_v, idx_v, gat_v, cont_v,
          row_v, sem, semb):
        wid = lax.axis_index("s") * info.num_cores + lax.axis_index("c")
        lane = lax.iota(jnp.int32, 16)
        stride_pat = lane * _F

        def sub_chunk(s, _):
            rbase = wid * nb + s * _N
            pltpu.sync_copy(cat_hbm.at[pl.ds(rbase * _F, _N * _F)], cat_v)
            ccopy = pltpu.async_copy(
                cont_hbm.at[pl.ds(rbase, _N), :], cont_v, sem
            )
            for f in range(_F):
                for g in range(_N // 16):
                    src = stride_pat + (g * 16 * _F + f)
                    vals = plsc.load_gather(cat_v, [src])
                    idx_v[f, pl.ds(g * 16, 16)] = vals + f * _V
            _HF = _F // 2
            copies = [
                pltpu.async_copy(
                    w_hbm.at[idx_v.at[f]],
                    gat_v.at[pl.ds(f * _N, _N), :],
                    sem if f < _HF else semb,
                )
                for f in range(_F)
            ]
            for c in copies[:_HF]:
                c.wait()
            ccopy.wait()

            def fill_row_a(r, _):
                row_v[r, pl.ds(0, 16)] = cont_v[r, :]
                for f in range(_HF):
                    row_v[r, pl.ds(_C + f * _D, _D)] = gat_v[f * _N + r, :]
                return ()

            lax.fori_loop(0, _N, fill_row_a, ())
            for c in copies[_HF:]:
                c.wait()

            def fill_row_b(r, _):
                for f in range(_HF, _F):
                    row_v[r, pl.ds(_C + f * _D, _D)] = gat_v[f * _N + r, :]
                return ()

            lax.fori_loop(0, _N, fill_row_b, ())
            pltpu.sync_copy(row_v, out_hbm.at[pl.ds(rbase, _N), :])
            return ()

        lax.fori_loop(0, n_sub, sub_chunk, ())

    return k


_repack = _make_repack_kernel()
_lookup = _make_lookup_kernel()


def kernel(x_cont, x_cat, W):
    w_t = jnp.swapaxes(W, 1, 2)
    w_tail = W[:, _VTAIL:, :]
    w_flat = _repack(w_t, w_tail)
    w2d = w_flat.reshape(_F * _V, _D)
    cat_flat = x_cat.reshape(-1)
    cont16 = jnp.pad(x_cont, ((0, 0), (0, 16 - _C)))
    return _lookup(cont16, cat_flat, w2d)


# speedup vs baseline: 1.0110x; 1.0110x over previous
import functools

import jax
import jax.numpy as jnp
from jax import lax
from jax.experimental import pallas as pl
from jax.experimental.pallas import tpu as pltpu
from jax.experimental.pallas import tpu_sc as plsc

_B = 16384
_F = 26
_V = 100000
_D = 16
_C = 13
_OUT = _C + _F * _D
_N = 128

_VT = 1536
_NFULL = _V // _VT
_VLAST = 99968 - _VT
_VTAIL = 99968
_NCHUNK = _NFULL + 1


def _make_repack_kernel():
    info = plsc.get_sparse_core_info()
    nw = info.num_cores * info.num_subcores
    n_units = _F * _NCHUNK

    mesh = plsc.VectorSubcoreMesh(core_axis_name="c", subcore_axis_name="s")

    @functools.partial(
        pl.kernel,
        mesh=mesh,
        out_type=jax.ShapeDtypeStruct((_F * _V * _D,), jnp.float32),
        scratch_types=[
            pltpu.VMEM((_D, _VT), jnp.float32),
            pltpu.VMEM((_D, _VT), jnp.float32),
            pltpu.VMEM((_VT * _D,), jnp.float32),
            pltpu.VMEM((_VT * _D,), jnp.float32),
            pltpu.VMEM((32, _D), jnp.float32),
            pltpu.SemaphoreType.DMA,
            pltpu.SemaphoreType.DMA,
            pltpu.SemaphoreType.DMA,
            pltpu.SemaphoreType.DMA,
        ],
        compiler_params=pltpu.CompilerParams(
            use_tc_tiling_on_sc=True, needs_layout_passes=False
        ),
    )
    def k(wt_hbm, tail_hbm, out_hbm, slab0, slab1, rows0, rows1, tail_v,
          isem0, isem1, osem0, osem1):
        wid = lax.axis_index("s") * info.num_cores + lax.axis_index("c")
        lane = lax.iota(jnp.int32, 16)
        lane16 = lane * _D
        n_mine = (n_units - 1 - wid) // nw + 1

        def unit_coords(t):
            u = wid + t * nw
            f = u // _NCHUNK
            ci = u % _NCHUNK
            vc = jnp.where(ci == _NFULL, _VLAST, ci * _VT)
            return f, vc

        def fire_in(t, slab, isem):
            f, vc = unit_coords(t)
            pltpu.async_copy(wt_hbm.at[f, :, pl.ds(vc, _VT)], slab, isem)

        fire_in(0, slab0, isem0)

        def unit(t, slab, rows, other_slab, isem, other_isem, osem):
            @pl.when(t < n_mine)
            def _():
                f, vc = unit_coords(t)
                pltpu.make_async_copy(
                    wt_hbm.at[0, :, pl.ds(0, _VT)], slab, isem
                ).wait()

                @pl.when(t + 1 < n_mine)
                def _():
                    fire_in(t + 1, other_slab, other_isem)

                @pl.when(t >= 2)
                def _():
                    pltpu.make_async_copy(
                        rows, out_hbm.at[pl.ds(0, _VT * _D)], osem
                    ).wait()

                def seg(s, _):
                    for half in range(2):
                        s2 = s * 2 + half
                        base = lane16 + s2 * (16 * _D)
                        for d in range(_D):
                            v16 = slab[d, pl.ds(s2 * 16, 16)]
                            plsc.store_scatter(rows, [base + d], v16)
                    return ()

                lax.fori_loop(0, _VT // 32, seg, ())
                pltpu.async_copy(
                    rows,
                    out_hbm.at[pl.ds((f * _V + vc) * _D, _VT * _D)],
                    osem,
                )

        def pair(p, _):
            unit(2 * p, slab0, rows0, slab1, isem0, isem1, osem0)
            unit(2 * p + 1, slab1, rows1, slab0, isem1, isem0, osem1)
            return ()

        lax.fori_loop(0, (n_units // nw + 2) // 2, pair, ())
        for rows, osem in ((rows0, osem0), (rows1, osem1)):
            pltpu.make_async_copy(
                rows, out_hbm.at[pl.ds(0, _VT * _D)], osem
            ).wait()

        @pl.when(wid < _F)
        def _():
            f = wid
            pltpu.sync_copy(tail_hbm.at[f], tail_v)

            def trow(i, _):
                for j in range(4):
                    r = i * 4 + j
                    rows0[pl.ds(r * _D, _D)] = tail_v[r, pl.ds(0, _D)]
                return ()

            lax.fori_loop(0, 8, trow, ())
            pltpu.sync_copy(
                rows0.at[pl.ds(0, 32 * _D)],
                out_hbm.at[pl.ds((f * _V + _VTAIL) * _D, 32 * _D)],
            )

    return k


def _make_lookup_kernel():
    info = plsc.get_sparse_core_info()
    nw = info.num_cores * info.num_subcores
    nb = _B // nw
    n_sub = nb // _N

    mesh = plsc.VectorSubcoreMesh(core_axis_name="c", subcore_axis_name="s")

    @functools.partial(
        pl.kernel,
        mesh=mesh,
        out_type=jax.ShapeDtypeStruct((_B, _OUT), jnp.float32),
        scratch_types=[
            pltpu.VMEM((_N * _F,), jnp.int32),
            pltpu.VMEM((_F, _N), jnp.int32),
            pltpu.VMEM((_F * _N, _D), jnp.float32),
            pltpu.VMEM((_N, 16), jnp.float32),
            pltpu.VMEM((_N, _OUT), jnp.float32),
            pltpu.SemaphoreType.DMA,
            pltpu.SemaphoreType.DMA,
            pltpu.SemaphoreType.DMA,
        ],
        compiler_params=pltpu.CompilerParams(
            use_tc_tiling_on_sc=False, needs_layout_passes=False
        ),
    )
    def k(cont_hbm, cat_hbm, w_hbm, out_hbm, cat_v, idx_v, gat_v, cont_v,
          row_v, sem, semb, osem):
        wid = lax.axis_index("s") * info.num_cores + lax.axis_index("c")
        lane = lax.iota(jnp.int32, 16)
        stride_pat = lane * _F

        def sub_chunk(s, _):
            rbase = wid * nb + s * _N
            pltpu.sync_copy(cat_hbm.at[pl.ds(rbase * _F, _N * _F)], cat_v)
            ccopy = pltpu.async_copy(
                cont_hbm.at[pl.ds(rbase, _N), :], cont_v, sem
            )
            for f in range(_F):
                for g in range(_N // 16):
                    src = stride_pat + (g * 16 * _F + f)
                    vals = plsc.load_gather(cat_v, [src])
                    idx_v[f, pl.ds(g * 16, 16)] = vals + f * _V
            _HF = _F // 2
            copies = [
                pltpu.async_copy(
                    w_hbm.at[idx_v.at[f]],
                    gat_v.at[pl.ds(f * _N, _N), :],
                    sem if f < _HF else semb,
                )
                for f in range(_F)
            ]
            for c in copies[:_HF]:
                c.wait()
            ccopy.wait()

            @pl.when(s > 0)
            def _():
                pltpu.make_async_copy(
                    row_v, out_hbm.at[pl.ds(0, _N), :], osem
                ).wait()

            def fill_row_a(r, _):
                row_v[r, pl.ds(0, 16)] = cont_v[r, :]
                for f in range(_HF):
                    row_v[r, pl.ds(_C + f * _D, _D)] = gat_v[f * _N + r, :]
                return ()

            lax.fori_loop(0, _N, fill_row_a, ())
            for c in copies[_HF:]:
                c.wait()

            def fill_row_b(r, _):
                for f in range(_HF, _F):
                    row_v[r, pl.ds(_C + f * _D, _D)] = gat_v[f * _N + r, :]
                return ()

            lax.fori_loop(0, _N, fill_row_b, ())
            pltpu.async_copy(row_v, out_hbm.at[pl.ds(rbase, _N), :], osem)
            return ()

        lax.fori_loop(0, n_sub, sub_chunk, ())
        pltpu.make_async_copy(row_v, out_hbm.at[pl.ds(0, _N), :], osem).wait()

    return k


_repack = _make_repack_kernel()
_lookup = _make_lookup_kernel()


def kernel(x_cont, x_cat, W):
    w_t = jnp.swapaxes(W, 1, 2)
    w_tail = W[:, _VTAIL:, :]
    w_flat = _repack(w_t, w_tail)
    w2d = w_flat.reshape(_F * _V, _D)
    cat_flat = x_cat.reshape(-1)
    cont16 = jnp.pad(x_cont, ((0, 0), (0, 16 - _C)))
    return _lookup(cont16, cat_flat, w2d)


# speedup vs baseline: 1.0114x; 1.0004x over previous
import functools

import jax
import jax.numpy as jnp
from jax import lax
from jax.experimental import pallas as pl
from jax.experimental.pallas import tpu as pltpu
from jax.experimental.pallas import tpu_sc as plsc

_B = 16384
_F = 26
_V = 100000
_D = 16
_C = 13
_OUT = _C + _F * _D
_N = 128

_VT = 1536
_NFULL = _V // _VT
_VLAST = 99968 - _VT
_VTAIL = 99968
_NCHUNK = _NFULL + 1


def _make_repack_kernel():
    info = plsc.get_sparse_core_info()
    nw = info.num_cores * info.num_subcores
    n_units = _F * _NCHUNK

    mesh = plsc.VectorSubcoreMesh(core_axis_name="c", subcore_axis_name="s")

    @functools.partial(
        pl.kernel,
        mesh=mesh,
        out_type=jax.ShapeDtypeStruct((_F * _V * _D,), jnp.float32),
        scratch_types=[
            pltpu.VMEM((_D, _VT), jnp.float32),
            pltpu.VMEM((_D, _VT), jnp.float32),
            pltpu.VMEM((_VT * _D,), jnp.float32),
            pltpu.VMEM((_VT * _D,), jnp.float32),
            pltpu.VMEM((32, _D), jnp.float32),
            pltpu.SemaphoreType.DMA,
            pltpu.SemaphoreType.DMA,
            pltpu.SemaphoreType.DMA,
            pltpu.SemaphoreType.DMA,
        ],
        compiler_params=pltpu.CompilerParams(
            use_tc_tiling_on_sc=True, needs_layout_passes=False
        ),
    )
    def k(wt_hbm, tail_hbm, out_hbm, slab0, slab1, rows0, rows1, tail_v,
          isem0, isem1, osem0, osem1):
        wid = lax.axis_index("s") * info.num_cores + lax.axis_index("c")
        lane = lax.iota(jnp.int32, 16)
        lane16 = lane * _D
        n_mine = (n_units - 1 - wid) // nw + 1

        def unit_coords(t):
            u = wid + t * nw
            f = u // _NCHUNK
            ci = u % _NCHUNK
            vc = jnp.where(ci == _NFULL, _VLAST, ci * _VT)
            return f, vc

        def fire_in(t, slab, isem):
            f, vc = unit_coords(t)
            pltpu.async_copy(
                wt_hbm.at[f, pl.ds(0, 8), pl.ds(vc, _VT)],
                slab.at[pl.ds(0, 8), :], isem,
            )
            pltpu.async_copy(
                wt_hbm.at[f, pl.ds(8, 8), pl.ds(vc, _VT)],
                slab.at[pl.ds(8, 8), :], isem,
            )

        fire_in(0, slab0, isem0)

        def unit(t, slab, rows, other_slab, isem, other_isem, osem):
            @pl.when(t < n_mine)
            def _():
                f, vc = unit_coords(t)
                for h in range(2):
                    pltpu.make_async_copy(
                        wt_hbm.at[0, pl.ds(h * 8, 8), pl.ds(0, _VT)],
                        slab.at[pl.ds(h * 8, 8), :], isem,
                    ).wait()

                @pl.when(t + 1 < n_mine)
                def _():
                    fire_in(t + 1, other_slab, other_isem)

                @pl.when(t >= 2)
                def _():
                    pltpu.make_async_copy(
                        rows, out_hbm.at[pl.ds(0, _VT * _D)], osem
                    ).wait()

                def seg(s, _):
                    for half in range(2):
                        s2 = s * 2 + half
                        base = lane16 + s2 * (16 * _D)
                        for d in range(_D):
                            v16 = slab[d, pl.ds(s2 * 16, 16)]
                            plsc.store_scatter(rows, [base + d], v16)
                    return ()

                lax.fori_loop(0, _VT // 32, seg, ())
                pltpu.async_copy(
                    rows,
                    out_hbm.at[pl.ds((f * _V + vc) * _D, _VT * _D)],
                    osem,
                )

        def pair(p, _):
            unit(2 * p, slab0, rows0, slab1, isem0, isem1, osem0)
            unit(2 * p + 1, slab1, rows1, slab0, isem1, isem0, osem1)
            return ()

        lax.fori_loop(0, (n_units // nw + 2) // 2, pair, ())
        for rows, osem in ((rows0, osem0), (rows1, osem1)):
            pltpu.make_async_copy(
                rows, out_hbm.at[pl.ds(0, _VT * _D)], osem
            ).wait()

        @pl.when(wid < _F)
        def _():
            f = wid
            pltpu.sync_copy(tail_hbm.at[f], tail_v)

            def trow(i, _):
                for j in range(4):
                    r = i * 4 + j
                    rows0[pl.ds(r * _D, _D)] = tail_v[r, pl.ds(0, _D)]
                return ()

            lax.fori_loop(0, 8, trow, ())
            pltpu.sync_copy(
                rows0.at[pl.ds(0, 32 * _D)],
                out_hbm.at[pl.ds((f * _V + _VTAIL) * _D, 32 * _D)],
            )

    return k


def _make_lookup_kernel():
    info = plsc.get_sparse_core_info()
    nw = info.num_cores * info.num_subcores
    nb = _B // nw
    n_sub = nb // _N

    mesh = plsc.VectorSubcoreMesh(core_axis_name="c", subcore_axis_name="s")

    @functools.partial(
        pl.kernel,
        mesh=mesh,
        out_type=jax.ShapeDtypeStruct((_B, _OUT), jnp.float32),
        scratch_types=[
            pltpu.VMEM((_N * _F,), jnp.int32),
            pltpu.VMEM((_F, _N), jnp.int32),
            pltpu.VMEM((_F * _N, _D), jnp.float32),
            pltpu.VMEM((_N, 16), jnp.float32),
            pltpu.VMEM((_N, _OUT), jnp.float32),
            pltpu.SemaphoreType.DMA,
            pltpu.SemaphoreType.DMA,
            pltpu.SemaphoreType.DMA,
        ],
        compiler_params=pltpu.CompilerParams(
            use_tc_tiling_on_sc=False, needs_layout_passes=False
        ),
    )
    def k(cont_hbm, cat_hbm, w_hbm, out_hbm, cat_v, idx_v, gat_v, cont_v,
          row_v, sem, semb, osem):
        wid = lax.axis_index("s") * info.num_cores + lax.axis_index("c")
        lane = lax.iota(jnp.int32, 16)
        stride_pat = lane * _F

        def sub_chunk(s, _):
            rbase = wid * nb + s * _N
            pltpu.sync_copy(cat_hbm.at[pl.ds(rbase * _F, _N * _F)], cat_v)
            ccopy = pltpu.async_copy(
                cont_hbm.at[pl.ds(rbase, _N), :], cont_v, sem
            )
            for f in range(_F):
                for g in range(_N // 16):
                    src = stride_pat + (g * 16 * _F + f)
                    vals = plsc.load_gather(cat_v, [src])
                    idx_v[f, pl.ds(g * 16, 16)] = vals + f * _V
            _HF = _F // 2
            copies = [
                pltpu.async_copy(
                    w_hbm.at[idx_v.at[f]],
                    gat_v.at[pl.ds(f * _N, _N), :],
                    sem if f < _HF else semb,
                )
                for f in range(_F)
            ]
            for c in copies[:_HF]:
                c.wait()
            ccopy.wait()

            @pl.when(s > 0)
            def _():
                pltpu.make_async_copy(
                    row_v, out_hbm.at[pl.ds(0, _N), :], osem
                ).wait()

            def fill_row_a(r, _):
                row_v[r, pl.ds(0, 16)] = cont_v[r, :]
                for f in range(_HF):
                    row_v[r, pl.ds(_C + f * _D, _D)] = gat_v[f * _N + r, :]
                return ()

            lax.fori_loop(0, _N, fill_row_a, ())
            for c in copies[_HF:]:
                c.wait()

            def fill_row_b(r, _):
                for f in range(_HF, _F):
                    row_v[r, pl.ds(_C + f * _D, _D)] = gat_v[f * _N + r, :]
                return ()

            lax.fori_loop(0, _N, fill_row_b, ())
            pltpu.async_copy(row_v, out_hbm.at[pl.ds(rbase, _N), :], osem)
            return ()

        lax.fori_loop(0, n_sub, sub_chunk, ())
        pltpu.make_async_copy(row_v, out_hbm.at[pl.ds(0, _N), :], osem).wait()

    return k


_repack = _make_repack_kernel()
_lookup = _make_lookup_kernel()


def kernel(x_cont, x_cat, W):
    w_t = jnp.swapaxes(W, 1, 2)
    w_tail = W[:, _VTAIL:, :]
    w_flat = _repack(w_t, w_tail)
    w2d = w_flat.reshape(_F * _V, _D)
    cat_flat = x_cat.reshape(-1)
    cont16 = jnp.pad(x_cont, ((0, 0), (0, 16 - _C)))
    return _lookup(cont16, cat_flat, w2d)
